# Initial kernel scaffold; baseline (speedup 1.0000x reference)
import jax
import jax.numpy as jnp
from jax.experimental import pallas as pl


def kernel(x_variables, x_virtual_constraints, x_variables_batch, W1, b1, W2, b2):
    raise NotImplementedError("write your pallas kernel here")


# speedup vs baseline: 3.8284x; 3.8284x over previous
import functools

import jax
import jax.numpy as jnp
from jax import lax
from jax.experimental import pallas as pl
from jax.experimental.pallas import tpu as pltpu
from jax.experimental.pallas import tpu_sc as plsc

N = 320000
B = 10000
E = 128
H = 256
LANES = 16
NC = 2
NS = 16
ROWS = 128
UNITS = N // ROWS
S_HALF = B // NC
ACC_ROWS = 5008
TRASH1 = ACC_ROWS - 1
ROWS_PER_TILE = 312
LAST_ROWS = ACC_ROWS - (NS - 1) * ROWS_PER_TILE


def _sc_segment_sums(ids, x, zsum):
    mesh = plsc.VectorSubcoreMesh(core_axis_name="c", subcore_axis_name="s",
                                  num_cores=NC)

    @functools.partial(
        pl.kernel,
        mesh=mesh,
        out_type=[
            jax.ShapeDtypeStruct((NC, ACC_ROWS, E), jnp.float32),
            jax.ShapeDtypeStruct((NC, ACC_ROWS, E), jnp.float32),
        ],
        scratch_types=[
            pltpu.VMEM((ROWS,), jnp.int32),
            pltpu.VMEM((ROWS, E), jnp.float32),
            pltpu.VMEM((ROWS, E), jnp.float32),
            pltpu.VMEM((LANES,), jnp.int32),
            pltpu.VMEM_SHARED((ACC_ROWS, E), jnp.float32),
        ],
    )
    def k(ids_hbm, x_hbm, zsum_hbm, sums_hbm, cnts_hbm,
          idx_v, rows_v, ones_v, probe_v, acc_s):
        cid = lax.axis_index("c")
        sid = lax.axis_index("s")
        one16 = jnp.ones((LANES,), jnp.float32)

        def orow(r, carry):
            for c in range(E // LANES):
                ones_v[r, pl.ds(c * LANES, LANES)] = one16
            return carry

        lax.fori_loop(0, ROWS, orow, 0)

        @pl.when(sid == 0)
        def _():
            pltpu.sync_copy(zsum_hbm, acc_s)

        def bs_step(_, lohi):
            lo, hi = lohi
            mid = jnp.minimum((lo + hi) // 2, UNITS - 1)
            pltpu.sync_copy(ids_hbm.at[pl.ds(mid * ROWS, LANES)], probe_v)
            pv = probe_v[...]
            p = pv[0] >= S_HALF
            active = lo < hi
            new_lo = jnp.where(active & jnp.logical_not(p), mid + 1, lo)
            new_hi = jnp.where(active & p, mid, hi)
            return new_lo, new_hi

        lo, hi = lax.fori_loop(0, 12, bs_step, (jnp.int32(0), jnp.int32(UNITS)))
        u_hi = hi

        u_start = jnp.where(cid == 0, 0, jnp.maximum(u_hi - 1, 0))
        u_end = jnp.where(cid == 0, u_hi, UNITS)
        nloc = jnp.maximum((u_end - u_start - sid + NS - 1) // NS, 0)

        def load_idx(u):
            pltpu.sync_copy(ids_hbm.at[pl.ds(u * ROWS, ROWS)], idx_v)

            @pl.when(cid == 0)
            def _():
                for c in range(ROWS // LANES):
                    v = idx_v[pl.ds(c * LANES, LANES)]
                    idx_v[pl.ds(c * LANES, LANES)] = jnp.minimum(v, S_HALF)

            @pl.when(cid != 0)
            def _():
                for c in range(ROWS // LANES):
                    v = idx_v[pl.ds(c * LANES, LANES)]
                    idx_v[pl.ds(c * LANES, LANES)] = jnp.where(
                        v >= S_HALF, v - S_HALF, TRASH1)

        def writeout(dst_hbm):
            base = sid * ROWS_PER_TILE

            @pl.when(sid == NS - 1)
            def _():
                pltpu.sync_copy(acc_s.at[pl.ds(base, LAST_ROWS)],
                                dst_hbm.at[cid].at[pl.ds(base, LAST_ROWS)])

            @pl.when(sid != NS - 1)
            def _():
                pltpu.sync_copy(acc_s.at[pl.ds(base, ROWS_PER_TILE)],
                                dst_hbm.at[cid].at[pl.ds(base, ROWS_PER_TILE)])

        plsc.subcore_barrier()

        def unit1(j, carry):
            u = u_start + sid + NS * j
            load_idx(u)
            pltpu.sync_copy(x_hbm.at[pl.ds(u * ROWS, ROWS)], rows_v)
            pltpu.sync_copy(rows_v, acc_s.at[idx_v], add=True)
            return carry

        lax.fori_loop(0, nloc, unit1, 0)
        plsc.subcore_barrier()
        writeout(sums_hbm)
        plsc.subcore_barrier()

        @pl.when(sid == 0)
        def _():
            pltpu.sync_copy(zsum_hbm, acc_s)

        plsc.subcore_barrier()

        def unit2(j, carry):
            u = u_start + sid + NS * j
            load_idx(u)
            pltpu.sync_copy(ones_v, acc_s.at[idx_v], add=True)
            return carry

        lax.fori_loop(0, nloc, unit2, 0)
        plsc.subcore_barrier()
        writeout(cnts_hbm)

    return k(ids, x, zsum)


def _tc_mlp(sums3, cnts3, xvc, w1a, w1b, b1, w2, b2):
    BS = 1000
    grid = (B // BS,)
    nb = S_HALF // BS

    def body(sums_ref, cnt_ref, xvc_ref, w1a_ref, w1b_ref, b1_ref, w2_ref,
             b2_ref, out_ref):
        s = sums_ref[0]
        c = cnt_ref[0, :, 0:1]
        m = s / jnp.maximum(c, 1.0)
        h = (jnp.dot(m, w1a_ref[...], preferred_element_type=jnp.float32,
                     precision=lax.Precision.HIGHEST)
             + jnp.dot(xvc_ref[...], w1b_ref[...],
                       preferred_element_type=jnp.float32,
                       precision=lax.Precision.HIGHEST)
             + b1_ref[...])
        h = jnp.maximum(h, 0.0)
        out_ref[...] = (jnp.dot(h, w2_ref[...],
                                preferred_element_type=jnp.float32,
                                precision=lax.Precision.HIGHEST)
                        + b2_ref[...])

    return pl.pallas_call(
        body,
        grid=grid,
        in_specs=[
            pl.BlockSpec((1, BS, E), lambda i: (i // nb, i % nb, 0)),
            pl.BlockSpec((1, BS, E), lambda i: (i // nb, i % nb, 0)),
            pl.BlockSpec((BS, E), lambda i: (i, 0)),
            pl.BlockSpec((E, H), lambda i: (0, 0)),
            pl.BlockSpec((E, H), lambda i: (0, 0)),
            pl.BlockSpec((1, H), lambda i: (0, 0)),
            pl.BlockSpec((H, E), lambda i: (0, 0)),
            pl.BlockSpec((1, E), lambda i: (0, 0)),
        ],
        out_specs=pl.BlockSpec((BS, E), lambda i: (i, 0)),
        out_shape=jax.ShapeDtypeStruct((B, E), jnp.float32),
    )(sums3, cnts3, xvc, w1a, w1b, b1, w2, b2)


def kernel(x_variables, x_virtual_constraints, x_variables_batch, W1, b1, W2, b2):
    ids = x_variables_batch.astype(jnp.int32)
    zsum = jnp.zeros((ACC_ROWS, E), jnp.float32)
    sums3, cnts3 = _sc_segment_sums(ids, x_variables, zsum)
    w1a = W1[:E]
    w1b = W1[E:]
    return _tc_mlp(sums3, cnts3, x_virtual_constraints, w1a, w1b,
                   b1.reshape(1, H), W2, b2.reshape(1, E))


# speedup vs baseline: 5.2659x; 1.3755x over previous
import functools

import jax
import jax.numpy as jnp
from jax import lax
from jax.experimental import pallas as pl
from jax.experimental.pallas import tpu as pltpu
from jax.experimental.pallas import tpu_sc as plsc

N = 320000
B = 10000
E = 128
H = 256
LANES = 16
NC = 2
NS = 16
ROWS = 128
UNITS = N // ROWS
S_HALF = B // NC
ACC_ROWS = 5008
TRASH1 = ACC_ROWS - 1
ROWS_PER_TILE = 312
LAST_ROWS = ACC_ROWS - (NS - 1) * ROWS_PER_TILE


def _sc_segment_sums(ids, x, zsum):
    mesh = plsc.VectorSubcoreMesh(core_axis_name="c", subcore_axis_name="s",
                                  num_cores=NC)

    @functools.partial(
        pl.kernel,
        mesh=mesh,
        out_type=[
            jax.ShapeDtypeStruct((NC, ACC_ROWS, E), jnp.float32),
            jax.ShapeDtypeStruct((NC, ACC_ROWS, E), jnp.float32),
        ],
        scratch_types=[
            pltpu.VMEM((ROWS,), jnp.int32),
            pltpu.VMEM((ROWS,), jnp.int32),
            pltpu.VMEM((ROWS, E), jnp.float32),
            pltpu.VMEM((ROWS, E), jnp.float32),
            pltpu.VMEM((ROWS, E), jnp.float32),
            pltpu.VMEM((LANES,), jnp.int32),
            pltpu.VMEM_SHARED((ACC_ROWS, E), jnp.float32),
            pltpu.SemaphoreType.DMA,
            pltpu.SemaphoreType.DMA,
            pltpu.SemaphoreType.DMA,
            pltpu.SemaphoreType.DMA,
            pltpu.SemaphoreType.DMA,
            pltpu.SemaphoreType.DMA,
        ],
    )
    def k(ids_hbm, x_hbm, zsum_hbm, sums_hbm, cnts_hbm,
          idx0_v, idx1_v, rows0_v, rows1_v, ones_v, probe_v, acc_s,
          si0, si1, sr0, sr1, ss0, ss1):
        idx_b = (idx0_v, idx1_v)
        rows_b = (rows0_v, rows1_v)
        si_b = (si0, si1)
        sr_b = (sr0, sr1)
        ss_b = (ss0, ss1)
        cid = lax.axis_index("c")
        sid = lax.axis_index("s")
        one16 = jnp.ones((LANES,), jnp.float32)

        def orow(r, carry):
            for c in range(E // LANES):
                ones_v[r, pl.ds(c * LANES, LANES)] = one16
            return carry

        lax.fori_loop(0, ROWS, orow, 0)

        @pl.when(sid == 0)
        def _():
            pltpu.sync_copy(zsum_hbm, acc_s)

        def bs_step(_, lohi):
            lo, hi = lohi
            mid = jnp.minimum((lo + hi) // 2, UNITS - 1)
            pltpu.sync_copy(ids_hbm.at[pl.ds(mid * ROWS, LANES)], probe_v)
            pv = probe_v[...]
            p = pv[0] >= S_HALF
            active = lo < hi
            new_lo = jnp.where(active & jnp.logical_not(p), mid + 1, lo)
            new_hi = jnp.where(active & p, mid, hi)
            return new_lo, new_hi

        lo, hi = lax.fori_loop(0, 12, bs_step, (jnp.int32(0), jnp.int32(UNITS)))
        u_hi = hi

        u_start = jnp.where(cid == 0, 0, jnp.maximum(u_hi - 1, 0))
        u_end = jnp.where(cid == 0, u_hi, UNITS)
        nloc = jnp.maximum((u_end - u_start - sid + NS - 1) // NS, 0)

        def transform_idx(idx_v):
            @pl.when(cid == 0)
            def _():
                for c in range(ROWS // LANES):
                    v = idx_v[pl.ds(c * LANES, LANES)]
                    idx_v[pl.ds(c * LANES, LANES)] = jnp.minimum(v, S_HALF)

            @pl.when(cid != 0)
            def _():
                for c in range(ROWS // LANES):
                    v = idx_v[pl.ds(c * LANES, LANES)]
                    idx_v[pl.ds(c * LANES, LANES)] = jnp.where(
                        v >= S_HALF, v - S_HALF, TRASH1)

        def writeout(dst_hbm):
            base = sid * ROWS_PER_TILE

            @pl.when(sid == NS - 1)
            def _():
                pltpu.sync_copy(acc_s.at[pl.ds(base, LAST_ROWS)],
                                dst_hbm.at[cid].at[pl.ds(base, LAST_ROWS)])

            @pl.when(sid != NS - 1)
            def _():
                pltpu.sync_copy(acc_s.at[pl.ds(base, ROWS_PER_TILE)],
                                dst_hbm.at[cid].at[pl.ds(base, ROWS_PER_TILE)])

        def phase(with_rows):
            def u_of(j):
                return u_start + sid + NS * j

            def stage(j, k):
                u = u_of(j)
                pltpu.async_copy(ids_hbm.at[pl.ds(u * ROWS, ROWS)],
                                 idx_b[k], si_b[k])
                if with_rows:
                    pltpu.async_copy(x_hbm.at[pl.ds(u * ROWS, ROWS)],
                                     rows_b[k], sr_b[k])

            def wait_idx(k):
                pltpu.make_async_copy(ids_hbm.at[pl.ds(0, ROWS)],
                                      idx_b[k], si_b[k]).wait()

            def wait_rows(k):
                pltpu.make_async_copy(x_hbm.at[pl.ds(0, ROWS)],
                                      rows_b[k], sr_b[k]).wait()

            def wait_scatter(k):
                pltpu.make_async_copy(x_hbm.at[pl.ds(0, ROWS)],
                                      rows_b[k], ss_b[k]).wait()

            @pl.when(nloc > 0)
            def _():
                stage(0, 0)

            def pair(g, carry):
                for k in (0, 1):
                    j = 2 * g + k

                    @pl.when(j < nloc)
                    def _():
                        wait_idx(k)
                        transform_idx(idx_b[k])
                        if with_rows:
                            wait_rows(k)

                        @pl.when((j + 1 < nloc) & (j >= 1))
                        def _():
                            wait_scatter(1 - k)

                        @pl.when(j + 1 < nloc)
                        def _():
                            stage(j + 1, 1 - k)

                        src = rows_b[k] if with_rows else ones_v
                        pltpu.async_copy(src, acc_s.at[idx_b[k]], ss_b[k],
                                         add=True)

                return carry

            lax.fori_loop(0, (nloc + 1) // 2, pair, 0)

            for k in (0, 1):
                @pl.when((nloc >= 1) & ((nloc - 1) % 2 == k))
                def _():
                    wait_scatter(k)

                @pl.when((nloc >= 2) & ((nloc - 2) % 2 == k))
                def _():
                    wait_scatter(k)

        plsc.subcore_barrier()

        phase(with_rows=True)
        plsc.subcore_barrier()
        writeout(sums_hbm)
        plsc.subcore_barrier()

        @pl.when(sid == 0)
        def _():
            pltpu.sync_copy(zsum_hbm, acc_s)

        plsc.subcore_barrier()

        phase(with_rows=False)
        plsc.subcore_barrier()
        writeout(cnts_hbm)

    return k(ids, x, zsum)


def _tc_mlp(sums3, cnts3, xvc, w1a, w1b, b1, w2, b2):
    BS = 1000
    grid = (B // BS,)
    nb = S_HALF // BS

    def body(sums_ref, cnt_ref, xvc_ref, w1a_ref, w1b_ref, b1_ref, w2_ref,
             b2_ref, out_ref):
        s = sums_ref[0]
        c = cnt_ref[0, :, 0:1]
        m = s / jnp.maximum(c, 1.0)
        h = (jnp.dot(m, w1a_ref[...], preferred_element_type=jnp.float32,
                     precision=lax.Precision.HIGHEST)
             + jnp.dot(xvc_ref[...], w1b_ref[...],
                       preferred_element_type=jnp.float32,
                       precision=lax.Precision.HIGHEST)
             + b1_ref[...])
        h = jnp.maximum(h, 0.0)
        out_ref[...] = (jnp.dot(h, w2_ref[...],
                                preferred_element_type=jnp.float32,
                                precision=lax.Precision.HIGHEST)
                        + b2_ref[...])

    return pl.pallas_call(
        body,
        grid=grid,
        in_specs=[
            pl.BlockSpec((1, BS, E), lambda i: (i // nb, i % nb, 0)),
            pl.BlockSpec((1, BS, E), lambda i: (i // nb, i % nb, 0)),
            pl.BlockSpec((BS, E), lambda i: (i, 0)),
            pl.BlockSpec((E, H), lambda i: (0, 0)),
            pl.BlockSpec((E, H), lambda i: (0, 0)),
            pl.BlockSpec((1, H), lambda i: (0, 0)),
            pl.BlockSpec((H, E), lambda i: (0, 0)),
            pl.BlockSpec((1, E), lambda i: (0, 0)),
        ],
        out_specs=pl.BlockSpec((BS, E), lambda i: (i, 0)),
        out_shape=jax.ShapeDtypeStruct((B, E), jnp.float32),
    )(sums3, cnts3, xvc, w1a, w1b, b1, w2, b2)


def kernel(x_variables, x_virtual_constraints, x_variables_batch, W1, b1, W2, b2):
    ids = x_variables_batch.astype(jnp.int32)
    zsum = jnp.zeros((ACC_ROWS, E), jnp.float32)
    sums3, cnts3 = _sc_segment_sums(ids, x_variables, zsum)
    w1a = W1[:E]
    w1b = W1[E:]
    return _tc_mlp(sums3, cnts3, x_virtual_constraints, w1a, w1b,
                   b1.reshape(1, H), W2, b2.reshape(1, E))


# speedup vs baseline: 5.2911x; 1.0048x over previous
import functools

import jax
import jax.numpy as jnp
from jax import lax
from jax.experimental import pallas as pl
from jax.experimental.pallas import tpu as pltpu
from jax.experimental.pallas import tpu_sc as plsc

N = 320000
B = 10000
E = 128
H = 256
LANES = 16
NC = 2
NS = 16
ROWS = 128
UNITS = N // ROWS
S_HALF = B // NC
ACC_ROWS = 5008
TRASH1 = ACC_ROWS - 1
ROWS_PER_TILE = 312
LAST_ROWS = ACC_ROWS - (NS - 1) * ROWS_PER_TILE


def _sc_segment_sums(ids, x, zsum):
    mesh = plsc.VectorSubcoreMesh(core_axis_name="c", subcore_axis_name="s",
                                  num_cores=NC)

    @functools.partial(
        pl.kernel,
        mesh=mesh,
        out_type=[
            jax.ShapeDtypeStruct((NC, ACC_ROWS, E), jnp.float32),
            jax.ShapeDtypeStruct((NC, ACC_ROWS, E), jnp.float32),
        ],
        scratch_types=[
            pltpu.VMEM((ROWS,), jnp.int32),
            pltpu.VMEM((ROWS,), jnp.int32),
            pltpu.VMEM((ROWS, E), jnp.float32),
            pltpu.VMEM((ROWS, E), jnp.float32),
            pltpu.VMEM((ROWS, E), jnp.float32),
            pltpu.VMEM((LANES,), jnp.int32),
            pltpu.VMEM_SHARED((ACC_ROWS, E), jnp.float32),
            pltpu.SemaphoreType.DMA,
            pltpu.SemaphoreType.DMA,
            pltpu.SemaphoreType.DMA,
            pltpu.SemaphoreType.DMA,
            pltpu.SemaphoreType.DMA,
            pltpu.SemaphoreType.DMA,
        ],
    )
    def k(ids_hbm, x_hbm, zsum_hbm, sums_hbm, cnts_hbm,
          idx0_v, idx1_v, rows0_v, rows1_v, ones_v, probe_v, acc_s,
          si0, si1, sr0, sr1, ss0, ss1):
        idx_b = (idx0_v, idx1_v)
        rows_b = (rows0_v, rows1_v)
        si_b = (si0, si1)
        sr_b = (sr0, sr1)
        ss_b = (ss0, ss1)
        cid = lax.axis_index("c")
        sid = lax.axis_index("s")
        one16 = jnp.ones((LANES,), jnp.float32)

        def orow(r, carry):
            for c in range(E // LANES):
                ones_v[r, pl.ds(c * LANES, LANES)] = one16
            return carry

        lax.fori_loop(0, ROWS, orow, 0)

        def zero_acc():
            base = sid * ROWS_PER_TILE

            @pl.when(sid == NS - 1)
            def _():
                pltpu.sync_copy(zsum_hbm.at[pl.ds(base, LAST_ROWS)],
                                acc_s.at[pl.ds(base, LAST_ROWS)])

            @pl.when(sid != NS - 1)
            def _():
                pltpu.sync_copy(zsum_hbm.at[pl.ds(base, ROWS_PER_TILE)],
                                acc_s.at[pl.ds(base, ROWS_PER_TILE)])

        zero_acc()

        def bs_step(_, lohi):
            lo, hi = lohi
            mid = jnp.minimum((lo + hi) // 2, UNITS - 1)
            pltpu.sync_copy(ids_hbm.at[pl.ds(mid * ROWS, LANES)], probe_v)
            pv = probe_v[...]
            p = pv[0] >= S_HALF
            active = lo < hi
            new_lo = jnp.where(active & jnp.logical_not(p), mid + 1, lo)
            new_hi = jnp.where(active & p, mid, hi)
            return new_lo, new_hi

        lo, hi = lax.fori_loop(0, 12, bs_step, (jnp.int32(0), jnp.int32(UNITS)))
        u_hi = hi

        u_start = jnp.where(cid == 0, 0, jnp.maximum(u_hi - 1, 0))
        u_end = jnp.where(cid == 0, u_hi, UNITS)
        nloc = jnp.maximum((u_end - u_start - sid + NS - 1) // NS, 0)

        def transform_idx(idx_v):
            @pl.when(cid == 0)
            def _():
                for c in range(ROWS // LANES):
                    v = idx_v[pl.ds(c * LANES, LANES)]
                    idx_v[pl.ds(c * LANES, LANES)] = jnp.minimum(v, S_HALF)

            @pl.when(cid != 0)
            def _():
                for c in range(ROWS // LANES):
                    v = idx_v[pl.ds(c * LANES, LANES)]
                    idx_v[pl.ds(c * LANES, LANES)] = jnp.where(
                        v >= S_HALF, v - S_HALF, TRASH1)

        def writeout(dst_hbm):
            base = sid * ROWS_PER_TILE

            @pl.when(sid == NS - 1)
            def _():
                pltpu.sync_copy(acc_s.at[pl.ds(base, LAST_ROWS)],
                                dst_hbm.at[cid].at[pl.ds(base, LAST_ROWS)])

            @pl.when(sid != NS - 1)
            def _():
                pltpu.sync_copy(acc_s.at[pl.ds(base, ROWS_PER_TILE)],
                                dst_hbm.at[cid].at[pl.ds(base, ROWS_PER_TILE)])

        def phase(with_rows):
            def u_of(j):
                return u_start + sid + NS * j

            def stage(j, k):
                u = u_of(j)
                pltpu.async_copy(ids_hbm.at[pl.ds(u * ROWS, ROWS)],
                                 idx_b[k], si_b[k])
                if with_rows:
                    pltpu.async_copy(x_hbm.at[pl.ds(u * ROWS, ROWS)],
                                     rows_b[k], sr_b[k])

            def wait_idx(k):
                pltpu.make_async_copy(ids_hbm.at[pl.ds(0, ROWS)],
                                      idx_b[k], si_b[k]).wait()

            def wait_rows(k):
                pltpu.make_async_copy(x_hbm.at[pl.ds(0, ROWS)],
                                      rows_b[k], sr_b[k]).wait()

            def wait_scatter(k):
                pltpu.make_async_copy(x_hbm.at[pl.ds(0, ROWS)],
                                      rows_b[k], ss_b[k]).wait()

            @pl.when(nloc > 0)
            def _():
                stage(0, 0)

            def pair(g, carry):
                for k in (0, 1):
                    j = 2 * g + k

                    @pl.when(j < nloc)
                    def _():
                        wait_idx(k)
                        transform_idx(idx_b[k])
                        if with_rows:
                            wait_rows(k)

                        @pl.when((j + 1 < nloc) & (j >= 1))
                        def _():
                            wait_scatter(1 - k)

                        @pl.when(j + 1 < nloc)
                        def _():
                            stage(j + 1, 1 - k)

                        src = rows_b[k] if with_rows else ones_v
                        pltpu.async_copy(src, acc_s.at[idx_b[k]], ss_b[k],
                                         add=True)

                return carry

            lax.fori_loop(0, (nloc + 1) // 2, pair, 0)

            for k in (0, 1):
                @pl.when((nloc >= 1) & ((nloc - 1) % 2 == k))
                def _():
                    wait_scatter(k)

                @pl.when((nloc >= 2) & ((nloc - 2) % 2 == k))
                def _():
                    wait_scatter(k)

        plsc.subcore_barrier()

        phase(with_rows=True)
        plsc.subcore_barrier()
        writeout(sums_hbm)
        plsc.subcore_barrier()
        zero_acc()
        plsc.subcore_barrier()

        phase(with_rows=False)
        plsc.subcore_barrier()
        writeout(cnts_hbm)

    return k(ids, x, zsum)


def _tc_mlp(sums3, cnts3, xvc, w1a, w1b, b1, w2, b2):
    BS = 1000
    grid = (B // BS,)
    nb = S_HALF // BS

    def body(sums_ref, cnt_ref, xvc_ref, w1a_ref, w1b_ref, b1_ref, w2_ref,
             b2_ref, out_ref):
        s = sums_ref[0]
        c = cnt_ref[0, :, 0:1]
        m = s / jnp.maximum(c, 1.0)
        h = (jnp.dot(m, w1a_ref[...], preferred_element_type=jnp.float32,
                     precision=lax.Precision.HIGHEST)
             + jnp.dot(xvc_ref[...], w1b_ref[...],
                       preferred_element_type=jnp.float32,
                       precision=lax.Precision.HIGHEST)
             + b1_ref[...])
        h = jnp.maximum(h, 0.0)
        out_ref[...] = (jnp.dot(h, w2_ref[...],
                                preferred_element_type=jnp.float32,
                                precision=lax.Precision.HIGHEST)
                        + b2_ref[...])

    return pl.pallas_call(
        body,
        grid=grid,
        in_specs=[
            pl.BlockSpec((1, BS, E), lambda i: (i // nb, i % nb, 0)),
            pl.BlockSpec((1, BS, E), lambda i: (i // nb, i % nb, 0)),
            pl.BlockSpec((BS, E), lambda i: (i, 0)),
            pl.BlockSpec((E, H), lambda i: (0, 0)),
            pl.BlockSpec((E, H), lambda i: (0, 0)),
            pl.BlockSpec((1, H), lambda i: (0, 0)),
            pl.BlockSpec((H, E), lambda i: (0, 0)),
            pl.BlockSpec((1, E), lambda i: (0, 0)),
        ],
        out_specs=pl.BlockSpec((BS, E), lambda i: (i, 0)),
        out_shape=jax.ShapeDtypeStruct((B, E), jnp.float32),
    )(sums3, cnts3, xvc, w1a, w1b, b1, w2, b2)


def kernel(x_variables, x_virtual_constraints, x_variables_batch, W1, b1, W2, b2):
    ids = x_variables_batch.astype(jnp.int32)
    zsum = jnp.zeros((ACC_ROWS, E), jnp.float32)
    sums3, cnts3 = _sc_segment_sums(ids, x_variables, zsum)
    w1a = W1[:E]
    w1b = W1[E:]
    return _tc_mlp(sums3, cnts3, x_virtual_constraints, w1a, w1b,
                   b1.reshape(1, H), W2, b2.reshape(1, E))
